# Initial kernel scaffold; baseline (speedup 1.0000x reference)
import jax
import jax.numpy as jnp
from jax.experimental import pallas as pl


def kernel(x, edge_attr, atom_tables, bond_tables):
    raise NotImplementedError("write your pallas kernel here")


# speedup vs baseline: 9.1819x; 9.1819x over previous
import functools

import jax
import jax.numpy as jnp
from jax import lax
from jax.experimental import pallas as pl

EMB = 128


def _emb_block(idx_ref, tbl_ref, out_ref, *, offsets):
    idx = idx_ref[...]
    b, j = idx.shape
    k = tbl_ref.shape[0]
    acc = jnp.zeros((b, k), dtype=jnp.float32)
    for jj in range(j):
        shifted = idx[:, jj : jj + 1] + offsets[jj]
        cols = lax.broadcasted_iota(jnp.int32, (b, k), 1)
        acc = acc + (cols == shifted).astype(jnp.float32)
    out_ref[...] = jnp.dot(acc, tbl_ref[...], preferred_element_type=jnp.float32)


def _lookup_sum(idx, tables, block_rows):
    n, j = idx.shape
    sizes = [int(t.shape[0]) for t in tables]
    offsets = [0] * j
    for jj in range(1, j):
        offsets[jj] = offsets[jj - 1] + sizes[jj - 1]
    k = sum(sizes)
    tbl = jnp.concatenate(tables, axis=0)
    grid = (n // block_rows,)
    return pl.pallas_call(
        functools.partial(_emb_block, offsets=offsets),
        grid=grid,
        in_specs=[
            pl.BlockSpec((block_rows, j), lambda i: (i, 0)),
            pl.BlockSpec((k, EMB), lambda i: (0, 0)),
        ],
        out_specs=pl.BlockSpec((block_rows, EMB), lambda i: (i, 0)),
        out_shape=jax.ShapeDtypeStruct((n, EMB), jnp.float32),
    )(idx, tbl)


def kernel(x, edge_attr, atom_tables, bond_tables):
    h = _lookup_sum(x, atom_tables, block_rows=2000)
    e = _lookup_sum(edge_attr, bond_tables, block_rows=8000)
    return (h, e)
